# baseline (speedup 1.0000x reference)
import functools

import jax
import jax.numpy as jnp
from jax import lax
from jax.experimental import pallas as pl
from jax.experimental.pallas import tpu as pltpu
from jax.experimental.pallas import tpu_sc as plsc

VOCAB = 1000000
DIM = 64
BATCH = 16384
NEG = 20

NC = 2
NS = 16
NW = NC * NS
BPW = BATCH // NW
C = 16
NCHUNK = BPW // C
NPAIR = NCHUNK // 2
KROWS = C * NEG
O_UH = KROWS
O_VP = 2 * KROWS
O_VH = 2 * KROWS + C
SLAB = 2 * KROWS + 2 * C
SLABP = 1024
OUTP = 1024

W = 256
NBLKF = VOCAB // W
TAILC = VOCAB - NBLKF * W


def _transpose_body(ut_hbm, vt_hbm, utail_hbm, vtail_hbm, u2_hbm, v2_hbm,
                    inb0, inb1, outb0, outb1, sin0, sin1, sout0, sout1):
    wid = lax.axis_index("s") * NC + lax.axis_index("c")
    lane = lax.iota(jnp.int32, 16)
    inbufs = (inb0, inb1)
    outbufs = (outb0, outb1)
    sins = (sin0, sin1)
    souts = (sout0, sout1)
    tid = wid & 15

    def make(table_hbm, tail_hbm, out_hbm):
        def fire_in(k, p):
            @pl.when(k < NBLKF)
            def _():
                for dt in range(DIM // 8):
                    pltpu.async_copy(
                        table_hbm.at[pl.ds(8 * dt, 8), pl.ds(k * W, W)],
                        inbufs[p].at[pl.ds(8 * dt, 8)], sins[p])

        def transpose_block(k, p):
            @pl.when(k < NBLKF)
            def _():
                pltpu.make_async_copy(table_hbm.at[:, pl.ds(0, W)],
                                      inbufs[p], sins[p]).wait()

                @pl.when(k >= 2 * 16)
                def _():
                    pltpu.make_async_copy(outbufs[p],
                                          out_hbm.at[pl.ds(0, W // 2)],
                                          souts[p]).wait()

                inb = inbufs[p]
                outb = outbufs[p]

                def row(j, _):
                    for c0 in range(0, 2 * DIM, 16):
                        h = c0 // DIM
                        d0 = c0 % DIM
                        col = jnp.full((16,), 2 * j + h, jnp.int32)
                        outb[j, pl.ds(c0, 16)] = plsc.load_gather(
                            inb, [lane + d0, col])
                    return 0

                lax.fori_loop(0, W // 2, row, 0)
                pltpu.async_copy(outb, out_hbm.at[pl.ds(k * (W // 2), W // 2)],
                                 souts[p])

        fire_in(tid, 0)
        @pl.when(tid == 0)
        def _():
            pltpu.sync_copy(tail_hbm, outbufs[0].at[pl.ds(0, TAILC // 2)])
            pltpu.sync_copy(outbufs[0].at[pl.ds(0, TAILC // 2)],
                            out_hbm.at[pl.ds(NBLKF * (W // 2), TAILC // 2)])

        def step(t, _):
            k0 = tid + 32 * t
            k1 = k0 + 16
            fire_in(k1, 1)
            transpose_block(k0, 0)
            fire_in(k0 + 32, 0)
            transpose_block(k1, 1)
            return 0

        lax.fori_loop(0, NBLKF // 32 + 1, step, 0)
        for p in range(2):
            pltpu.make_async_copy(outbufs[p], out_hbm.at[pl.ds(0, W // 2)],
                                  souts[p]).wait()


    @pl.when(wid < 16)
    def _():
        make(ut_hbm, utail_hbm, u2_hbm)

    @pl.when(wid >= 16)
    def _():
        make(vt_hbm, vtail_hbm, v2_hbm)


def _gather_body(u2_hbm, v2_hbm, idx_hbm, out_hbm,
                 pb0, pb1, ur0, ur1, vr0, vr1, ov0, ov1,
                 si0, si1, su0, su1, sv0, sv1, so0, so1):
    wid = lax.axis_index("s") * NC + lax.axis_index("c")
    lane = lax.iota(jnp.int32, 16)
    rbase = lane * NEG
    pairbufs = (pb0, pb1)
    urows = (ur0, ur1)
    vrows = (vr0, vr1)
    outvs = (ov0, ov1)
    sidx = (si0, si1)
    sus = (su0, su1)
    svs = (sv0, sv1)
    sos = (so0, so1)

    def fire_idxpair(k, pp):
        pltpu.async_copy(
            idx_hbm.at[pl.ds((wid * NPAIR + k) * 2 * SLABP, 2 * SLABP)],
            pairbufs[pp], sidx[pp])

    def wait_idxpair(pp):
        pltpu.make_async_copy(idx_hbm.at[pl.ds(0, 2 * SLABP)], pairbufs[pp],
                              sidx[pp]).wait()

    def fire_gathers(g, cp, pp, q):
        ib = pairbufs[pp]
        o = q * SLABP
        pltpu.async_copy(u2_hbm.at[ib.at[pl.ds(o, 128)]],
                         urows[cp].at[pl.ds(0, 128)], sus[cp])
        pltpu.async_copy(u2_hbm.at[ib.at[pl.ds(o + 128, 128)]],
                         urows[cp].at[pl.ds(128, 128)], sus[cp])
        pltpu.async_copy(u2_hbm.at[ib.at[pl.ds(o + 256, 64)]],
                         urows[cp].at[pl.ds(256, 64)], sus[cp])
        pltpu.async_copy(v2_hbm.at[ib.at[pl.ds(o + O_VP, C)]],
                         vrows[cp], svs[cp])

    def compute(g, cp, pp, q):
        pltpu.make_async_copy(u2_hbm.at[pl.ds(0, KROWS)], urows[cp],
                              sus[cp]).wait()
        pltpu.make_async_copy(v2_hbm.at[pl.ds(0, C)], vrows[cp],
                              svs[cp]).wait()

        @pl.when(g >= 2)
        def _():
            pltpu.make_async_copy(outvs[cp], out_hbm.at[pl.ds(0, OUTP)],
                                  sos[cp]).wait()

        ib = pairbufs[pp]
        ur = urows[cp]
        vr = vrows[cp]
        ov = outvs[cp]
        o = q * SLABP
        vcol0 = ib[pl.ds(o + O_VH, 16)] * DIM
        ucol0 = tuple(
            plsc.load_gather(ib, [o + O_UH + rbase + l]) * DIM
            for l in range(NEG))

        def dstep(d, acc):
            dvec = jnp.full((16,), d, jnp.int32)
            vv = plsc.load_gather(vr, [lane, vcol0 + dvec])
            return tuple(
                acc[l] + plsc.load_gather(
                    ur, [rbase + l, ucol0[l] + dvec]) * vv
                for l in range(NEG))

        acc = lax.fori_loop(
            0, DIM, dstep,
            tuple(jnp.zeros((16,), jnp.float32) for _ in range(NEG)))
        for l in range(NEG):
            plsc.store_scatter(ov, [rbase + l], acc[l])

        def sig(i, _):
            x = ov[pl.ds(i * 16, 16)]
            ov[pl.ds(i * 16, 16)] = 1.0 / (1.0 + jnp.exp(-x))
            return 0

        lax.fori_loop(0, KROWS // 16, sig, 0)
        pltpu.async_copy(ov,
                         out_hbm.at[pl.ds((wid * NCHUNK + g) * OUTP, OUTP)],
                         sos[cp])

    fire_idxpair(0, 0)
    wait_idxpair(0)
    fire_gathers(0, 0, 0, 0)
    fire_idxpair(1, 1)

    def step(t, _):
        ka = 2 * t
        kb = 2 * t + 1
        g0 = 2 * ka
        fire_gathers(g0 + 1, 1, 0, 1)
        compute(g0, 0, 0, 0)
        wait_idxpair(1)
        fire_gathers(2 * kb, 0, 1, 0)
        compute(g0 + 1, 1, 0, 1)

        @pl.when(t < NPAIR // 2 - 1)
        def _():
            fire_idxpair(ka + 2, 0)
        fire_gathers(2 * kb + 1, 1, 1, 1)
        compute(2 * kb, 0, 1, 0)

        @pl.when(t < NPAIR // 2 - 1)
        def _():
            wait_idxpair(0)
            fire_gathers(2 * (ka + 2), 0, 0, 0)
        compute(2 * kb + 1, 1, 1, 1)

        @pl.when(t < NPAIR // 2 - 1)
        def _():
            fire_idxpair(kb + 2, 1)
        return 0

    lax.fori_loop(0, NPAIR // 2, step, 0)
    for p in range(2):
        pltpu.make_async_copy(outvs[p], out_hbm.at[pl.ds(0, OUTP)],
                              sos[p]).wait()


@jax.jit
def _run(ut, vt, utail, vtail, idx):
    mesh = plsc.VectorSubcoreMesh(core_axis_name="c", subcore_axis_name="s")
    cp = pltpu.CompilerParams(
        needs_layout_passes=False, use_tc_tiling_on_sc=True)
    u2, v2 = pl.kernel(
        _transpose_body,
        out_type=(jax.ShapeDtypeStruct((VOCAB // 2, 2 * DIM), jnp.float32),
                  jax.ShapeDtypeStruct((VOCAB // 2, 2 * DIM), jnp.float32)),
        mesh=mesh,
        scratch_types=[
            pltpu.VMEM((DIM, W), jnp.float32),
            pltpu.VMEM((DIM, W), jnp.float32),
            pltpu.VMEM((W // 2, 2 * DIM), jnp.float32),
            pltpu.VMEM((W // 2, 2 * DIM), jnp.float32),
            pltpu.SemaphoreType.DMA,
            pltpu.SemaphoreType.DMA,
            pltpu.SemaphoreType.DMA,
            pltpu.SemaphoreType.DMA,
        ],
        compiler_params=cp,
    )(ut, vt, utail, vtail)
    out = pl.kernel(
        _gather_body,
        out_type=jax.ShapeDtypeStruct((NW * NCHUNK * OUTP,), jnp.float32),
        mesh=mesh,
        scratch_types=[
            pltpu.VMEM((2 * SLABP,), jnp.int32),
            pltpu.VMEM((2 * SLABP,), jnp.int32),
            pltpu.VMEM((KROWS, 2 * DIM), jnp.float32),
            pltpu.VMEM((KROWS, 2 * DIM), jnp.float32),
            pltpu.VMEM((C, 2 * DIM), jnp.float32),
            pltpu.VMEM((C, 2 * DIM), jnp.float32),
            pltpu.VMEM((OUTP,), jnp.float32),
            pltpu.VMEM((OUTP,), jnp.float32),
            pltpu.SemaphoreType.DMA,
            pltpu.SemaphoreType.DMA,
            pltpu.SemaphoreType.DMA,
            pltpu.SemaphoreType.DMA,
            pltpu.SemaphoreType.DMA,
            pltpu.SemaphoreType.DMA,
            pltpu.SemaphoreType.DMA,
            pltpu.SemaphoreType.DMA,
        ],
        compiler_params=cp,
    )(u2, v2, idx)
    return out


def kernel(u_embeds, v_embeds, centers, context_and_negatives):
    ut = u_embeds.T
    vt = v_embeds.T
    utail = u_embeds[NBLKF * W:].reshape(TAILC // 2, 2 * DIM)
    vtail = v_embeds[NBLKF * W:].reshape(TAILC // 2, 2 * DIM)
    cen = jnp.asarray(centers, jnp.int32).reshape(NW, NCHUNK, C)
    ctx = jnp.asarray(context_and_negatives, jnp.int32).reshape(
        NW, NCHUNK, KROWS)
    idx = jnp.concatenate([ctx >> 1, ctx & 1, cen >> 1, cen & 1],
                          axis=-1)
    idx = jnp.pad(idx, ((0, 0), (0, 0), (0, SLABP - SLAB))).reshape(-1)
    out = _run(ut, vt, utail, vtail, idx)
    return out.reshape(NW * NCHUNK, OUTP)[:, :KROWS].reshape(BATCH, NEG)


# speedup vs baseline: 2.5997x; 2.5997x over previous
import functools

import jax
import jax.numpy as jnp
from jax import lax
from jax.experimental import pallas as pl
from jax.experimental.pallas import tpu as pltpu
from jax.experimental.pallas import tpu_sc as plsc

VOCAB = 1000000
DIM = 64
BATCH = 16384
NEG = 20

NC = 2
NS = 16
NW = NC * NS
BPW = BATCH // NW
C = 16
NCHUNK = BPW // C
NPAIR = NCHUNK // 2
KROWS = C * NEG
O_V = KROWS
SLABP = 1024
OUTP = 1024


def _gather_body(u_hbm, v_hbm, idx_hbm, out_hbm,
                 pb0, pb1, ur0, ur1, vr0, vr1, ov0, ov1,
                 si0, si1, su0, su1, sv0, sv1, so0, so1):
    wid = lax.axis_index("s") * NC + lax.axis_index("c")
    lane = lax.iota(jnp.int32, 16)
    mask0 = lane == 0
    perms = tuple(lane ^ k for k in (8, 4, 2, 1))
    _dn = lax.GatherDimensionNumbers(
        offset_dims=(), collapsed_slice_dims=(0,), start_index_map=(0,))

    def _perm(x, pm):
        return lax.gather(x, pm[:, None], dimension_numbers=_dn,
                          slice_sizes=(1,),
                          mode=lax.GatherScatterMode.PROMISE_IN_BOUNDS)
    pairbufs = (pb0, pb1)
    urows = (ur0, ur1)
    vrows = (vr0, vr1)
    outvs = (ov0, ov1)
    sidx = (si0, si1)
    sus = (su0, su1)
    svs = (sv0, sv1)
    sos = (so0, so1)

    def fire_idxpair(k, pp):
        pltpu.async_copy(
            idx_hbm.at[pl.ds((wid * NPAIR + k) * 2 * SLABP, 2 * SLABP)],
            pairbufs[pp], sidx[pp])

    def wait_idxpair(pp):
        pltpu.make_async_copy(idx_hbm.at[pl.ds(0, 2 * SLABP)], pairbufs[pp],
                              sidx[pp]).wait()

    def fire_gathers(g, cp, pp, q):
        ib = pairbufs[pp]
        o = q * SLABP
        pltpu.async_copy(u_hbm.at[ib.at[pl.ds(o, 128)]],
                         urows[cp].at[pl.ds(0, 128)], sus[cp])
        pltpu.async_copy(u_hbm.at[ib.at[pl.ds(o + 128, 128)]],
                         urows[cp].at[pl.ds(128, 128)], sus[cp])
        pltpu.async_copy(u_hbm.at[ib.at[pl.ds(o + 256, 64)]],
                         urows[cp].at[pl.ds(256, 64)], sus[cp])
        pltpu.async_copy(v_hbm.at[ib.at[pl.ds(o + O_V, C)]],
                         vrows[cp], svs[cp])

    def compute(g, cp):
        pltpu.make_async_copy(u_hbm.at[pl.ds(0, KROWS)], urows[cp],
                              sus[cp]).wait()
        pltpu.make_async_copy(v_hbm.at[pl.ds(0, C)], vrows[cp],
                              svs[cp]).wait()

        @pl.when(g >= 2)
        def _():
            pltpu.make_async_copy(outvs[cp], out_hbm.at[pl.ds(0, OUTP)],
                                  sos[cp]).wait()

        ur = urows[cp]
        vr = vrows[cp]
        ov = outvs[cp]

        def brow(b, _):
            v0 = vr[b, pl.ds(0, 16)]
            v1 = vr[b, pl.ds(16, 16)]
            v2 = vr[b, pl.ds(32, 16)]
            v3 = vr[b, pl.ds(48, 16)]
            for l in range(NEG):
                r = b * NEG + l
                p = ur[r, pl.ds(0, 16)] * v0
                p = p + ur[r, pl.ds(16, 16)] * v1
                p = p + ur[r, pl.ds(32, 16)] * v2
                p = p + ur[r, pl.ds(48, 16)] * v3
                for pm in perms:
                    p = p + _perm(p, pm)
                plsc.store_scatter(ov, [jnp.full((16,), r, jnp.int32)],
                                   p, mask=mask0)
            return 0

        lax.fori_loop(0, C, brow, 0)

        def sig(i, _):
            x = ov[pl.ds(i * 16, 16)]
            ov[pl.ds(i * 16, 16)] = 1.0 / (1.0 + jnp.exp(-x))
            return 0

        lax.fori_loop(0, KROWS // 16, sig, 0)
        pltpu.async_copy(ov,
                         out_hbm.at[pl.ds((wid * NCHUNK + g) * OUTP, OUTP)],
                         sos[cp])

    fire_idxpair(0, 0)
    wait_idxpair(0)
    fire_gathers(0, 0, 0, 0)
    fire_idxpair(1, 1)

    def step(t, _):
        ka = 2 * t
        kb = 2 * t + 1
        g0 = 2 * ka
        fire_gathers(g0 + 1, 1, 0, 1)
        compute(g0, 0)
        wait_idxpair(1)
        fire_gathers(2 * kb, 0, 1, 0)
        compute(g0 + 1, 1)

        @pl.when(t < NPAIR // 2 - 1)
        def _():
            fire_idxpair(ka + 2, 0)
        fire_gathers(2 * kb + 1, 1, 1, 1)
        compute(2 * kb, 0)

        @pl.when(t < NPAIR // 2 - 1)
        def _():
            wait_idxpair(0)
            fire_gathers(2 * (ka + 2), 0, 0, 0)
        compute(2 * kb + 1, 1)

        @pl.when(t < NPAIR // 2 - 1)
        def _():
            fire_idxpair(kb + 2, 1)
        return 0

    lax.fori_loop(0, NPAIR // 2, step, 0)
    for p in range(2):
        pltpu.make_async_copy(outvs[p], out_hbm.at[pl.ds(0, OUTP)],
                              sos[p]).wait()


@jax.jit
def _run(u_embeds, v_embeds, idx):
    mesh = plsc.VectorSubcoreMesh(core_axis_name="c", subcore_axis_name="s")
    cp = pltpu.CompilerParams(
        needs_layout_passes=False, use_tc_tiling_on_sc=False)
    out = pl.kernel(
        _gather_body,
        out_type=jax.ShapeDtypeStruct((NW * NCHUNK * OUTP,), jnp.float32),
        mesh=mesh,
        scratch_types=[
            pltpu.VMEM((2 * SLABP,), jnp.int32),
            pltpu.VMEM((2 * SLABP,), jnp.int32),
            pltpu.VMEM((KROWS, DIM), jnp.float32),
            pltpu.VMEM((KROWS, DIM), jnp.float32),
            pltpu.VMEM((C, DIM), jnp.float32),
            pltpu.VMEM((C, DIM), jnp.float32),
            pltpu.VMEM((OUTP,), jnp.float32),
            pltpu.VMEM((OUTP,), jnp.float32),
            pltpu.SemaphoreType.DMA,
            pltpu.SemaphoreType.DMA,
            pltpu.SemaphoreType.DMA,
            pltpu.SemaphoreType.DMA,
            pltpu.SemaphoreType.DMA,
            pltpu.SemaphoreType.DMA,
            pltpu.SemaphoreType.DMA,
            pltpu.SemaphoreType.DMA,
        ],
        compiler_params=cp,
    )(u_embeds, v_embeds, idx)
    return out


def kernel(u_embeds, v_embeds, centers, context_and_negatives):
    cen = jnp.asarray(centers, jnp.int32).reshape(NW, NCHUNK, C)
    ctx = jnp.asarray(context_and_negatives, jnp.int32).reshape(
        NW, NCHUNK, KROWS)
    idx = jnp.concatenate([ctx, cen], axis=-1)
    idx = jnp.pad(idx, ((0, 0), (0, 0), (0, SLABP - KROWS - C))).reshape(-1)
    out = _run(u_embeds, v_embeds, idx)
    return out.reshape(NW * NCHUNK, OUTP)[:, :KROWS].reshape(BATCH, NEG)
